# baseline (speedup 1.0000x reference)
import functools

import jax
import jax.numpy as jnp
from jax import lax
from jax.experimental import pallas as pl
from jax.experimental.pallas import tpu as pltpu
from jax.experimental.pallas import tpu_sc as plsc

_ROWS_PER_BLOCK = 2048
_NW = 32
_SC_CHUNK = 32
_UNROLL = 8


def _tc_add_kernel(x_ref, e_ref, o_ref):
    o_ref[...] = x_ref[...] + e_ref[...]


def _sc_add_last_batch(x_flat_all, emb_flat, B, S, D):
    rows_per_w = S // _NW
    n_chunks = rows_per_w // _SC_CHUNK
    CW = _SC_CHUNK * D
    mesh = plsc.VectorSubcoreMesh(core_axis_name="c", subcore_axis_name="s")

    @functools.partial(
        pl.kernel,
        mesh=mesh,
        out_type=jax.ShapeDtypeStruct((S * D,), jnp.float32),
        scratch_types=[
            pltpu.VMEM((CW,), jnp.float32),
            pltpu.VMEM((CW,), jnp.float32),
        ],
    )
    def sc_add(x_hbm, e_hbm, o_hbm, xb, eb):
        wid = lax.axis_index("s") * 2 + lax.axis_index("c")
        srow = wid * rows_per_w
        xbase = (B - 1) * S

        def chunk_body(t, carry):
            row0 = srow + t * _SC_CHUNK
            pltpu.sync_copy(e_hbm.at[pl.ds(row0 * D, CW)], eb)
            pltpu.sync_copy(x_hbm.at[pl.ds((xbase + row0) * D, CW)], xb)

            def add_body(i, c):
                base = i * (16 * _UNROLL)
                for u in range(_UNROLL):
                    o = base + u * 16
                    xb[pl.ds(o, 16)] = xb[pl.ds(o, 16)] + eb[pl.ds(o, 16)]
                return c

            lax.fori_loop(0, CW // (16 * _UNROLL), add_body, 0)
            pltpu.sync_copy(xb, o_hbm.at[pl.ds(row0 * D, CW)])
            return carry

        lax.fori_loop(0, n_chunks, chunk_body, 0)

    return sc_add(x_flat_all, emb_flat)


def kernel(inputs, embeddings):
    B, S, D = inputs.shape
    bs = _ROWS_PER_BLOCK
    sblk = S // bs
    x = inputs.reshape(B * S, D)
    tc_out = pl.pallas_call(
        _tc_add_kernel,
        grid=(sblk, B),
        in_specs=[
            pl.BlockSpec((bs, D), lambda s, b: (b * sblk + s, 0)),
            pl.BlockSpec((bs, D), lambda s, b: (s, 0)),
        ],
        out_specs=pl.BlockSpec((bs, D), lambda s, b: (b * sblk + s, 0)),
        out_shape=jax.ShapeDtypeStruct((B * S, D), inputs.dtype),
    )(x, embeddings)
    sc_out = _sc_add_last_batch(
        inputs.reshape(B * S * D), embeddings.reshape(S * D), B, S, D
    )
    flat = tc_out.reshape(B * S * D)
    flat = lax.dynamic_update_slice(flat, sc_out[0:8], ((B - 1) * S * D,))
    return flat.reshape(B, S, D)


# speedup vs baseline: 5.6207x; 5.6207x over previous
import jax
import jax.numpy as jnp
from jax.experimental import pallas as pl
from jax.experimental.pallas import tpu as pltpu

_ROWS_PER_BLOCK = 2048


def _add_kernel(x_ref, e_ref, o_ref):
    o_ref[...] = x_ref[...] + e_ref[...]


def kernel(inputs, embeddings):
    B, S, D = inputs.shape
    bs = _ROWS_PER_BLOCK
    sblk = S // bs
    x = inputs.reshape(B * S, D)
    out = pl.pallas_call(
        _add_kernel,
        grid=(sblk, B),
        in_specs=[
            pl.BlockSpec((bs, D), lambda s, b: (b * sblk + s, 0)),
            pl.BlockSpec((bs, D), lambda s, b: (s, 0)),
        ],
        out_specs=pl.BlockSpec((bs, D), lambda s, b: (b * sblk + s, 0)),
        out_shape=jax.ShapeDtypeStruct((B * S, D), inputs.dtype),
        compiler_params=pltpu.CompilerParams(
            dimension_semantics=("parallel", "arbitrary")
        ),
    )(x, embeddings)
    return out.reshape(B, S, D)
